# Initial kernel scaffold; baseline (speedup 1.0000x reference)
import jax
import jax.numpy as jnp
from jax.experimental import pallas as pl


def kernel(q_pts, s_pts, s_feats, neighb_inds, W1, gamma, beta, W2, b2, kernel_points):
    raise NotImplementedError("write your pallas kernel here")


# speedup vs baseline: 1.9666x; 1.9666x over previous
import functools

import jax
import jax.numpy as jnp
from jax import lax
from jax.experimental import pallas as pl
from jax.experimental.pallas import tpu as pltpu
from jax.experimental.pallas import tpu_sc as plsc

N = 10000
H = 32
C = 128
K = 15
SIGMA = 1.0

NW = 32
QW = 320
N_PAD = NW * QW
GRP = 16
NGRP = QW // GRP

_EPS = 1e-5


def _stats_body(feats_ref, w1_ref, gamma_ref, beta_ref, ab_ref, g_ref, s_ref):
    i = pl.program_id(0)

    @pl.when(i == 0)
    def _init():
        g_ref[...] = jnp.zeros_like(g_ref)
        s_ref[...] = jnp.zeros_like(s_ref)

    x = feats_ref[...]
    g_ref[...] += lax.dot_general(x, x, (((0,), (0,)), ((), ())),
                                  preferred_element_type=jnp.float32)
    s_ref[...] += jnp.sum(x, axis=0, keepdims=True)

    @pl.when(i == pl.num_programs(0) - 1)
    def _fin():
        w1 = w1_ref[...]
        g = g_ref[...]
        s = s_ref[...]
        mean = (s @ w1) / N
        t = lax.dot_general(g, w1, (((1,), (0,)), ((), ())),
                            preferred_element_type=jnp.float32)
        ex2 = jnp.sum(w1 * t, axis=0, keepdims=True) / N
        var = ex2 - mean * mean
        a = gamma_ref[...] * lax.rsqrt(var + _EPS)
        b = beta_ref[...] - mean * a
        ab_ref[0:1, :] = a
        ab_ref[1:2, :] = b


def _tc_stats(s_feats, W1, gamma, beta):
    nb = 10
    blk = N // nb
    return pl.pallas_call(
        _stats_body,
        grid=(nb,),
        in_specs=[
            pl.BlockSpec((blk, C), lambda i: (i, 0)),
            pl.BlockSpec((C, C), lambda i: (0, 0)),
            pl.BlockSpec((1, C), lambda i: (0, 0)),
            pl.BlockSpec((1, C), lambda i: (0, 0)),
        ],
        out_specs=pl.BlockSpec((2, C), lambda i: (0, 0)),
        out_shape=jax.ShapeDtypeStruct((2, C), jnp.float32),
        scratch_shapes=[
            pltpu.VMEM((C, C), jnp.float32),
            pltpu.VMEM((1, C), jnp.float32),
        ],
    )(s_feats, W1, gamma.reshape(1, C), beta.reshape(1, C))


def _mlp_body(feats_ref, w1_ref, ab_ref, w2_ref, b2_ref, cw_ref):
    x = feats_ref[...] @ w1_ref[...]
    y = x * ab_ref[0:1, :] + ab_ref[1:2, :]
    y = jnp.where(y >= 0, y, 0.1 * y)
    cw_ref[...] = y @ w2_ref[...] + b2_ref[...]


def _tc_mlp(s_feats_pad, W1, ab, W2, b2):
    nb = 10
    blk = N_PAD // nb
    return pl.pallas_call(
        _mlp_body,
        grid=(nb,),
        in_specs=[
            pl.BlockSpec((blk, C), lambda i: (i, 0)),
            pl.BlockSpec((C, C), lambda i: (0, 0)),
            pl.BlockSpec((2, C), lambda i: (0, 0)),
            pl.BlockSpec((C, K * C), lambda i: (0, 0)),
            pl.BlockSpec((1, K * C), lambda i: (0, 0)),
        ],
        out_specs=pl.BlockSpec((blk, K * C), lambda i: (i, 0)),
        out_shape=jax.ShapeDtypeStruct((N_PAD, K * C), jnp.float32),
    )(s_feats_pad, W1, ab, W2, b2.reshape(1, K * C))


def _iota16():
    return lax.iota(jnp.int32, 16)


def _bc_i(v):
    return jnp.full((16,), v, jnp.int32)


def _sqrt16(x):
    i = plsc.bitcast(x, jnp.int32)
    y = plsc.bitcast(jnp.int32(0x5F3759DF) - lax.shift_right_arithmetic(i, 1),
                     jnp.float32)
    for _ in range(3):
        y = y * (1.5 - 0.5 * x * y * y)
    return x * y


def _sc_geom_body(spts_hbm, qpts_hbm, ind_hbm, kp_hbm, infl_hbm, kidx_hbm,
                  spts_v, qpts_v, ind_v, kp_v, infl_v, kidx_v):
    wid = lax.axis_index("s") * 2 + lax.axis_index("c")
    ii = _iota16()

    pltpu.sync_copy(spts_hbm, spts_v)
    pltpu.sync_copy(qpts_hbm.at[wid], qpts_v)
    pltpu.sync_copy(ind_hbm.at[wid], ind_v)
    pltpu.sync_copy(kp_hbm, kp_v)

    def group_body(g, _):
        qb = g * GRP
        qx = plsc.load_gather(qpts_v, [qb + ii])
        qy = plsc.load_gather(qpts_v, [QW + qb + ii])
        qz = plsc.load_gather(qpts_v, [2 * QW + qb + ii])

        def h_body(h, _):
            idx = plsc.load_gather(ind_v, [(qb + ii) * H + h])
            nx = plsc.load_gather(spts_v, [idx]) - qx
            ny = plsc.load_gather(spts_v, [N + idx]) - qy
            nz = plsc.load_gather(spts_v, [2 * N + idx]) - qz
            best = jnp.full((16,), 1e30, jnp.float32)
            amin = jnp.zeros((16,), jnp.int32)
            for k in range(K):
                kx = kp_v[pl.ds((3 * k + 0) * 16, 16)]
                ky = kp_v[pl.ds((3 * k + 1) * 16, 16)]
                kz = kp_v[pl.ds((3 * k + 2) * 16, 16)]
                dx = nx - kx
                dy = ny - ky
                dz = nz - kz
                d2 = dx * dx + dy * dy + dz * dz
                m = d2 < best
                best = jnp.where(m, d2, best)
                amin = jnp.where(m, k, amin)
            infl = jnp.maximum(1.0 - _sqrt16(best) / SIGMA, 0.0)
            plsc.store_scatter(infl_v, [(qb + ii) * H + h], infl)
            plsc.store_scatter(kidx_v, [(qb + ii) * H + h], amin)
            return 0

        lax.fori_loop(0, H, h_body, 0)
        return 0

    lax.fori_loop(0, NGRP, group_body, 0)
    pltpu.sync_copy(infl_v, infl_hbm.at[wid])
    pltpu.sync_copy(kidx_v, kidx_hbm.at[wid])


def _sc_agg_body(ind_hbm, infl_hbm, kidx_hbm, feats_hbm, cw_hbm, out_hbm,
                 ind_v, infl_v, kidx_v, rows_v, cwq_v, out_v, sem_r, sem_c):
    wid = lax.axis_index("s") * 2 + lax.axis_index("c")
    pltpu.sync_copy(ind_hbm.at[wid], ind_v)
    pltpu.sync_copy(infl_hbm.at[wid], infl_v)
    pltpu.sync_copy(kidx_hbm.at[wid], kidx_v)

    def q_body(ql, _):
        n_glob = wid * QW + ql
        cp_r = pltpu.async_copy(feats_hbm.at[ind_v.at[pl.ds(ql * H, H)]],
                                rows_v, sem_r)
        cp_c = pltpu.async_copy(cw_hbm.at[pl.ds(n_glob * (K * C), K * C)],
                                cwq_v, sem_c)
        cp_r.wait()
        cp_c.wait()
        iv = [infl_v[pl.ds(ql * H, 16)], infl_v[pl.ds(ql * H + 16, 16)]]
        kv = [kidx_v[pl.ds(ql * H, 16)], kidx_v[pl.ds(ql * H + 16, 16)]]
        acc = [jnp.zeros((16,), jnp.float32) for _ in range(8)]
        for h in range(H):
            infl_s = iv[h // 16][h % 16]
            base = kv[h // 16][h % 16] * C
            for c in range(8):
                rowv = rows_v[h, pl.ds(c * 16, 16)]
                cwv = cwq_v[pl.ds(base + c * 16, 16)]
                acc[c] = acc[c] + infl_s * rowv * cwv
        for c in range(8):
            out_v[pl.ds(ql * C + c * 16, 16)] = acc[c]
        return 0

    lax.fori_loop(0, QW, q_body, 0)
    pltpu.sync_copy(out_v, out_hbm.at[wid])


def _sc_gather(spts_t, qpts_w, ind_w, kp_pad, s_feats_pad, cw):
    mesh = plsc.VectorSubcoreMesh(core_axis_name="c", subcore_axis_name="s")
    cp = pltpu.CompilerParams(
        needs_layout_passes=False, use_tc_tiling_on_sc=False)
    geom = pl.kernel(
        _sc_geom_body,
        out_type=(jax.ShapeDtypeStruct((NW, QW * H), jnp.float32),
                  jax.ShapeDtypeStruct((NW, QW * H), jnp.int32)),
        mesh=mesh,
        compiler_params=cp,
        scratch_types=[
            pltpu.VMEM((3 * N,), jnp.float32),
            pltpu.VMEM((3 * QW,), jnp.float32),
            pltpu.VMEM((QW * H,), jnp.int32),
            pltpu.VMEM((3 * K * 16,), jnp.float32),
            pltpu.VMEM((QW * H,), jnp.float32),
            pltpu.VMEM((QW * H,), jnp.int32),
        ],
    )
    infl, kidx = geom(spts_t, qpts_w, ind_w, kp_pad)
    agg = pl.kernel(
        _sc_agg_body,
        out_type=jax.ShapeDtypeStruct((NW, QW * C), jnp.float32),
        mesh=mesh,
        compiler_params=cp,
        scratch_types=[
            pltpu.VMEM((QW * H,), jnp.int32),
            pltpu.VMEM((QW * H,), jnp.float32),
            pltpu.VMEM((QW * H,), jnp.int32),
            pltpu.VMEM((H, C), jnp.float32),
            pltpu.VMEM((K * C,), jnp.float32),
            pltpu.VMEM((QW * C,), jnp.float32),
            pltpu.SemaphoreType.DMA,
            pltpu.SemaphoreType.DMA,
        ],
    )
    return agg(ind_w, infl, kidx, s_feats_pad, cw)


@jax.jit
def kernel(q_pts, s_pts, s_feats, neighb_inds, W1, gamma, beta, W2, b2,
           kernel_points):
    ab = _tc_stats(s_feats, W1, gamma, beta)
    s_feats_pad = jnp.concatenate(
        [s_feats, jnp.zeros((N_PAD - N, C), jnp.float32)], axis=0)
    cw = _tc_mlp(s_feats_pad, W1, ab, W2, b2).reshape(N_PAD * K * C)

    spts_t = s_pts.T.reshape(3 * N)
    qpts_pad = jnp.concatenate(
        [q_pts, jnp.zeros((N_PAD - N, 3), jnp.float32)], axis=0)
    qpts_w = qpts_pad.reshape(NW, QW, 3).transpose(0, 2, 1).reshape(NW, 3 * QW)
    ind_pad = jnp.concatenate(
        [neighb_inds.astype(jnp.int32),
         jnp.zeros((N_PAD - N, H), jnp.int32)], axis=0)
    ind_w = ind_pad.reshape(NW, QW * H)
    kp_pad = jnp.repeat(kernel_points.reshape(3 * K), 16)

    out = _sc_gather(spts_t, qpts_w, ind_w, kp_pad, s_feats_pad, cw)
    return out.reshape(N_PAD, C)[:N]


# speedup vs baseline: 2.8815x; 1.4652x over previous
import functools

import jax
import jax.numpy as jnp
from jax import lax
from jax.experimental import pallas as pl
from jax.experimental.pallas import tpu as pltpu
from jax.experimental.pallas import tpu_sc as plsc

N = 10000
H = 32
C = 128
K = 15
SIGMA = 1.0

NW = 32
QW = 320
N_PAD = NW * QW
GRP = 16
NGRP = QW // GRP

_EPS = 1e-5


def _stats_body(feats_ref, w1_ref, gamma_ref, beta_ref, ab_ref, g_ref, s_ref):
    i = pl.program_id(0)

    @pl.when(i == 0)
    def _init():
        g_ref[...] = jnp.zeros_like(g_ref)
        s_ref[...] = jnp.zeros_like(s_ref)

    x = feats_ref[...]
    g_ref[...] += lax.dot_general(x, x, (((0,), (0,)), ((), ())),
                                  preferred_element_type=jnp.float32)
    s_ref[...] += jnp.sum(x, axis=0, keepdims=True)

    @pl.when(i == pl.num_programs(0) - 1)
    def _fin():
        w1 = w1_ref[...]
        g = g_ref[...]
        s = s_ref[...]
        mean = (s @ w1) / N
        t = lax.dot_general(g, w1, (((1,), (0,)), ((), ())),
                            preferred_element_type=jnp.float32)
        ex2 = jnp.sum(w1 * t, axis=0, keepdims=True) / N
        var = ex2 - mean * mean
        a = gamma_ref[...] * lax.rsqrt(var + _EPS)
        b = beta_ref[...] - mean * a
        ab_ref[0:1, :] = a
        ab_ref[1:2, :] = b


def _tc_stats(s_feats, W1, gamma, beta):
    nb = 10
    blk = N // nb
    return pl.pallas_call(
        _stats_body,
        grid=(nb,),
        in_specs=[
            pl.BlockSpec((blk, C), lambda i: (i, 0)),
            pl.BlockSpec((C, C), lambda i: (0, 0)),
            pl.BlockSpec((1, C), lambda i: (0, 0)),
            pl.BlockSpec((1, C), lambda i: (0, 0)),
        ],
        out_specs=pl.BlockSpec((2, C), lambda i: (0, 0)),
        out_shape=jax.ShapeDtypeStruct((2, C), jnp.float32),
        scratch_shapes=[
            pltpu.VMEM((C, C), jnp.float32),
            pltpu.VMEM((1, C), jnp.float32),
        ],
    )(s_feats, W1, gamma.reshape(1, C), beta.reshape(1, C))


def _mlp_body(feats_ref, w1_ref, ab_ref, w2_ref, b2_ref, cw_ref):
    x = feats_ref[...] @ w1_ref[...]
    y = x * ab_ref[0:1, :] + ab_ref[1:2, :]
    y = jnp.where(y >= 0, y, 0.1 * y)
    cw_ref[...] = y @ w2_ref[...] + b2_ref[...]


def _tc_mlp(s_feats_pad, W1, ab, W2, b2):
    nb = 10
    blk = N_PAD // nb
    return pl.pallas_call(
        _mlp_body,
        grid=(nb,),
        in_specs=[
            pl.BlockSpec((blk, C), lambda i: (i, 0)),
            pl.BlockSpec((C, C), lambda i: (0, 0)),
            pl.BlockSpec((2, C), lambda i: (0, 0)),
            pl.BlockSpec((C, K * C), lambda i: (0, 0)),
            pl.BlockSpec((1, K * C), lambda i: (0, 0)),
        ],
        out_specs=pl.BlockSpec((blk, K * C), lambda i: (i, 0)),
        out_shape=jax.ShapeDtypeStruct((N_PAD, K * C), jnp.float32),
    )(s_feats_pad, W1, ab, W2, b2.reshape(1, K * C))


def _iota16():
    return lax.iota(jnp.int32, 16)


def _bc_i(v):
    return jnp.full((16,), v, jnp.int32)


def _sqrt16(x):
    i = plsc.bitcast(x, jnp.int32)
    y = plsc.bitcast(jnp.int32(0x5F3759DF) - lax.shift_right_arithmetic(i, 1),
                     jnp.float32)
    for _ in range(3):
        y = y * (1.5 - 0.5 * x * y * y)
    return x * y


def _sc_geom_body(spts_hbm, qpts_hbm, ind_hbm, kp_hbm, infl_hbm, kidx_hbm,
                  spts_v, qpts_v, ind_v, kp_v, infl_v, kidx_v):
    wid = lax.axis_index("s") * 2 + lax.axis_index("c")
    ii = _iota16()

    pltpu.sync_copy(spts_hbm, spts_v)
    pltpu.sync_copy(qpts_hbm.at[wid], qpts_v)
    pltpu.sync_copy(ind_hbm.at[wid], ind_v)
    pltpu.sync_copy(kp_hbm, kp_v)

    def group_body(g, _):
        qb = g * GRP
        qx = plsc.load_gather(qpts_v, [qb + ii])
        qy = plsc.load_gather(qpts_v, [QW + qb + ii])
        qz = plsc.load_gather(qpts_v, [2 * QW + qb + ii])

        def h_body(h, _):
            idx = plsc.load_gather(ind_v, [(qb + ii) * H + h])
            nx = plsc.load_gather(spts_v, [idx]) - qx
            ny = plsc.load_gather(spts_v, [N + idx]) - qy
            nz = plsc.load_gather(spts_v, [2 * N + idx]) - qz
            best = jnp.full((16,), 1e30, jnp.float32)
            amin = jnp.zeros((16,), jnp.int32)
            for k in range(K):
                kx = kp_v[pl.ds((3 * k + 0) * 16, 16)]
                ky = kp_v[pl.ds((3 * k + 1) * 16, 16)]
                kz = kp_v[pl.ds((3 * k + 2) * 16, 16)]
                dx = nx - kx
                dy = ny - ky
                dz = nz - kz
                d2 = dx * dx + dy * dy + dz * dz
                m = d2 < best
                best = jnp.where(m, d2, best)
                amin = jnp.where(m, k, amin)
            infl = jnp.maximum(1.0 - _sqrt16(best) / SIGMA, 0.0)
            plsc.store_scatter(infl_v, [(qb + ii) * H + h], infl)
            plsc.store_scatter(kidx_v, [(qb + ii) * H + h], amin)
            return 0

        lax.fori_loop(0, H, h_body, 0)
        return 0

    lax.fori_loop(0, NGRP, group_body, 0)
    pltpu.sync_copy(infl_v, infl_hbm.at[wid])
    pltpu.sync_copy(kidx_v, kidx_hbm.at[wid])


def _sc_agg_body(ind_hbm, infl_hbm, kidx_hbm, feats_hbm, cw_hbm, out_hbm,
                 ind_v, infl_v, kidx_v, rows0_v, rows1_v, cwq0_v, cwq1_v,
                 out_v, sr0, sc0, sr1, sc1):
    wid = lax.axis_index("s") * 2 + lax.axis_index("c")
    pltpu.sync_copy(ind_hbm.at[wid], ind_v)
    pltpu.sync_copy(infl_hbm.at[wid], infl_v)
    pltpu.sync_copy(kidx_hbm.at[wid], kidx_v)

    def start_q(ql, rows_ref, cwq_ref, sr, sc_):
        n_glob = wid * QW + ql
        pltpu.async_copy(feats_hbm.at[ind_v.at[pl.ds(ql * H, H)]],
                         rows_ref, sr)
        pltpu.async_copy(cw_hbm.at[pl.ds(n_glob * (K * C), K * C)],
                         cwq_ref, sc_)

    def wait_q(rows_ref, cwq_ref, sr, sc_):
        pltpu.make_async_copy(feats_hbm.at[ind_v.at[pl.ds(0, H)]],
                              rows_ref, sr).wait()
        pltpu.make_async_copy(cw_hbm.at[pl.ds(0, K * C)],
                              cwq_ref, sc_).wait()

    def compute_q(ql, rows_ref, cwq_ref):
        iv = [infl_v[pl.ds(ql * H, 16)], infl_v[pl.ds(ql * H + 16, 16)]]
        kv = [kidx_v[pl.ds(ql * H, 16)], kidx_v[pl.ds(ql * H + 16, 16)]]
        acc = [jnp.zeros((16,), jnp.float32) for _ in range(8)]
        for h in range(H):
            infl_s = iv[h // 16][h % 16]
            base = kv[h // 16][h % 16] * C
            for c in range(8):
                rowv = rows_ref[h, pl.ds(c * 16, 16)]
                cwv = cwq_ref[pl.ds(base + c * 16, 16)]
                acc[c] = acc[c] + infl_s * rowv * cwv
        for c in range(8):
            out_v[pl.ds(ql * C + c * 16, 16)] = acc[c]

    start_q(0, rows0_v, cwq0_v, sr0, sc0)

    def pair_body(i, _):
        ql0 = 2 * i
        ql1 = 2 * i + 1
        start_q(ql1, rows1_v, cwq1_v, sr1, sc1)
        wait_q(rows0_v, cwq0_v, sr0, sc0)
        compute_q(ql0, rows0_v, cwq0_v)
        qln = jnp.where(ql1 + 1 < QW, ql1 + 1, 0)
        start_q(qln, rows0_v, cwq0_v, sr0, sc0)
        wait_q(rows1_v, cwq1_v, sr1, sc1)
        compute_q(ql1, rows1_v, cwq1_v)
        return 0

    lax.fori_loop(0, QW // 2, pair_body, 0)
    wait_q(rows0_v, cwq0_v, sr0, sc0)
    pltpu.sync_copy(out_v, out_hbm.at[wid])


def _sc_gather(spts_t, qpts_w, ind_w, kp_pad, s_feats_pad, cw):
    mesh = plsc.VectorSubcoreMesh(core_axis_name="c", subcore_axis_name="s")
    cp = pltpu.CompilerParams(
        needs_layout_passes=False, use_tc_tiling_on_sc=False)
    geom = pl.kernel(
        _sc_geom_body,
        out_type=(jax.ShapeDtypeStruct((NW, QW * H), jnp.float32),
                  jax.ShapeDtypeStruct((NW, QW * H), jnp.int32)),
        mesh=mesh,
        compiler_params=cp,
        scratch_types=[
            pltpu.VMEM((3 * N,), jnp.float32),
            pltpu.VMEM((3 * QW,), jnp.float32),
            pltpu.VMEM((QW * H,), jnp.int32),
            pltpu.VMEM((3 * K * 16,), jnp.float32),
            pltpu.VMEM((QW * H,), jnp.float32),
            pltpu.VMEM((QW * H,), jnp.int32),
        ],
    )
    infl, kidx = geom(spts_t, qpts_w, ind_w, kp_pad)
    agg = pl.kernel(
        _sc_agg_body,
        out_type=jax.ShapeDtypeStruct((NW, QW * C), jnp.float32),
        mesh=mesh,
        compiler_params=cp,
        scratch_types=[
            pltpu.VMEM((QW * H,), jnp.int32),
            pltpu.VMEM((QW * H,), jnp.float32),
            pltpu.VMEM((QW * H,), jnp.int32),
            pltpu.VMEM((H, C), jnp.float32),
            pltpu.VMEM((H, C), jnp.float32),
            pltpu.VMEM((K * C,), jnp.float32),
            pltpu.VMEM((K * C,), jnp.float32),
            pltpu.VMEM((QW * C,), jnp.float32),
            pltpu.SemaphoreType.DMA,
            pltpu.SemaphoreType.DMA,
            pltpu.SemaphoreType.DMA,
            pltpu.SemaphoreType.DMA,
        ],
    )
    return agg(ind_w, infl, kidx, s_feats_pad, cw)


@jax.jit
def kernel(q_pts, s_pts, s_feats, neighb_inds, W1, gamma, beta, W2, b2,
           kernel_points):
    ab = _tc_stats(s_feats, W1, gamma, beta)
    s_feats_pad = jnp.concatenate(
        [s_feats, jnp.zeros((N_PAD - N, C), jnp.float32)], axis=0)
    cw = _tc_mlp(s_feats_pad, W1, ab, W2, b2).reshape(N_PAD * K * C)

    spts_t = s_pts.T.reshape(3 * N)
    qpts_pad = jnp.concatenate(
        [q_pts, jnp.zeros((N_PAD - N, 3), jnp.float32)], axis=0)
    qpts_w = qpts_pad.reshape(NW, QW, 3).transpose(0, 2, 1).reshape(NW, 3 * QW)
    ind_pad = jnp.concatenate(
        [neighb_inds.astype(jnp.int32),
         jnp.zeros((N_PAD - N, H), jnp.int32)], axis=0)
    ind_w = ind_pad.reshape(NW, QW * H)
    kp_pad = jnp.repeat(kernel_points.reshape(3 * K), 16)

    out = _sc_gather(spts_t, qpts_w, ind_w, kp_pad, s_feats_pad, cw)
    return out.reshape(N_PAD, C)[:N]
